# Initial kernel scaffold; baseline (speedup 1.0000x reference)
import jax
import jax.numpy as jnp
from jax.experimental import pallas as pl


def kernel(x, table, W, b):
    raise NotImplementedError("write your pallas kernel here")


# speedup vs baseline: 6.8595x; 6.8595x over previous
import jax
import jax.numpy as jnp
from jax import lax
from jax.experimental import pallas as pl
from jax.experimental.pallas import tpu as pltpu
from jax.experimental.pallas import tpu_sc as plsc

_B = 16384
_H = 200
_D = 16
_V = 1000000
_NC = 2
_NS = 16
_NW = _NC * _NS

_G_TOT = _V // 16
_GPW = 1954
_CHG = 32
_NCHV = 62

_RW = _B // _NW
_CH = 64
_NCH = _RW // _CH


def _t_body(tab_hbm, w_hbm, t_hbm, wv, tab_v, tv, sem):
    wid = lax.axis_index("s") * _NC + lax.axis_index("c")
    pltpu.sync_copy(w_hbm, wv)
    wvec = wv[...]
    ws = [wvec[d] for d in range(_D)]
    iota16 = lax.iota(jnp.int32, 16)

    def chunk(k, carry):
        g0 = jnp.minimum(wid * _GPW + k * _CHG, _G_TOT - _CHG)
        r0 = g0 * 16
        pltpu.sync_copy(tab_hbm.at[pl.ds(r0, _CHG * 16)], tab_v)

        def group(j, carry2):
            row0 = j * 16
            acc = jnp.zeros((16,), jnp.float32)
            for d in range(_D):
                col = plsc.load_gather(
                    tab_v, [row0 + iota16, jnp.full((16,), d, jnp.int32)])
                acc = acc + col * ws[d]
            tv[pl.ds(row0, 16)] = acc
            return carry2

        lax.fori_loop(0, _CHG, group, 0)
        pltpu.sync_copy(tv, t_hbm.at[pl.ds(r0, _CHG * 16)])
        return carry

    lax.fori_loop(0, _NCHV, chunk, 0)


def _gather_body(x_hbm, t_hbm, p_hbm, idx_v, vals_v, acc_v, sem):
    wid = lax.axis_index("s") * _NC + lax.axis_index("c")
    base0 = wid * _RW
    mask_hi = jnp.where(lax.iota(jnp.int32, 16) >= 8, 1.0, 0.0)

    def chunk(ci, carry):
        base = base0 + ci * _CH
        pltpu.sync_copy(x_hbm.at[pl.ds(base * _H, _CH * _H)], idx_v)
        pltpu.async_copy(t_hbm.at[idx_v], vals_v, sem).wait()

        def row(r, carry2):
            o = r * _H
            acc = vals_v[pl.ds(o + 184, 16)] * mask_hi
            for k in range(12):
                acc = acc + vals_v[pl.ds(o + k * 16, 16)]
            acc_v[r, :] = acc
            return carry2

        lax.fori_loop(0, _CH, row, 0)
        pltpu.sync_copy(acc_v, p_hbm.at[pl.ds(base, _CH)])
        return carry

    lax.fori_loop(0, _NCH, chunk, 0)


def _final_body(b_ref, p_ref, out_ref):
    s = jnp.sum(p_ref[...], axis=1, keepdims=True)
    z = s * (1.0 / _H) + b_ref[0]
    y = 1.0 / (1.0 + jnp.exp(-z))
    out_ref[...] = jnp.round(y * 10000.0) / 10000.0


def kernel(x, table, W, b):
    t = pl.kernel(
        _t_body,
        out_type=jax.ShapeDtypeStruct((_V,), jnp.float32),
        mesh=plsc.VectorSubcoreMesh(core_axis_name="c", subcore_axis_name="s"),
        compiler_params=pltpu.CompilerParams(needs_layout_passes=False),
        scratch_types=[
            pltpu.VMEM((_D,), jnp.float32),
            pltpu.VMEM((_CHG * 16, _D), jnp.float32),
            pltpu.VMEM((_CHG * 16,), jnp.float32),
            pltpu.SemaphoreType.DMA,
        ],
    )(table, W.reshape(_D))

    part = pl.kernel(
        _gather_body,
        out_type=jax.ShapeDtypeStruct((_B, _D), jnp.float32),
        mesh=plsc.VectorSubcoreMesh(core_axis_name="c", subcore_axis_name="s"),
        compiler_params=pltpu.CompilerParams(needs_layout_passes=False),
        scratch_types=[
            pltpu.VMEM((_CH * _H,), jnp.int32),
            pltpu.VMEM((_CH * _H,), jnp.float32),
            pltpu.VMEM((_CH, _D), jnp.float32),
            pltpu.SemaphoreType.DMA,
        ],
    )(x.reshape(_B * _H), t)

    out = pl.pallas_call(
        _final_body,
        grid=(8,),
        in_specs=[
            pl.BlockSpec(memory_space=pltpu.SMEM),
            pl.BlockSpec((_B // 8, _D), lambda i: (i, 0)),
        ],
        out_specs=pl.BlockSpec((_B // 8, 1), lambda i: (i, 0)),
        out_shape=jax.ShapeDtypeStruct((_B, 1), jnp.float32),
    )(b, part)
    return out


# speedup vs baseline: 7.6114x; 1.1096x over previous
import jax
import jax.numpy as jnp
from jax import lax
from jax.experimental import pallas as pl
from jax.experimental.pallas import tpu as pltpu
from jax.experimental.pallas import tpu_sc as plsc

_B = 16384
_H = 200
_D = 16
_V = 1000000
_NC = 2
_NS = 16
_NW = _NC * _NS

_G_TOT = _V // 16
_GPW = 1954
_CHG = 32
_NCHV = 62

_RW = _B // _NW
_CH = 64
_NCH = _RW // _CH


def _t_body(tab_hbm, w_hbm, t_hbm, wv, tab_v, tv, sem):
    wid = lax.axis_index("s") * _NC + lax.axis_index("c")
    pltpu.sync_copy(w_hbm, wv)
    wvec = wv[...]
    ws = [wvec[d] for d in range(_D)]
    iota16 = lax.iota(jnp.int32, 16)

    def chunk(k, carry):
        g0 = jnp.minimum(wid * _GPW + k * _CHG, _G_TOT - _CHG)
        r0 = g0 * 16
        pltpu.sync_copy(tab_hbm.at[pl.ds(r0, _CHG * 16)], tab_v)

        def group(j, carry2):
            row0 = j * 16
            acc = jnp.zeros((16,), jnp.float32)
            for d in range(_D):
                col = plsc.load_gather(
                    tab_v, [row0 + iota16, jnp.full((16,), d, jnp.int32)])
                acc = acc + col * ws[d]
            tv[pl.ds(row0, 16)] = acc
            return carry2

        lax.fori_loop(0, _CHG, group, 0)
        pltpu.sync_copy(tv, t_hbm.at[pl.ds(r0, _CHG * 16)])
        return carry

    lax.fori_loop(0, _NCHV, chunk, 0)


_TSH = 62528


def _gather_body(x_hbm, t_hbm, p_hbm, idx_v, vals_v, acc_v, bounce_v, tsh, sem):
    wid = lax.axis_index("s") * _NC + lax.axis_index("c")
    base0 = wid * _RW
    mask_hi = jnp.where(lax.iota(jnp.int32, 16) >= 8, 1.0, 0.0)

    sid = lax.axis_index("s")
    st = jnp.minimum(sid * _TSH, _V - _TSH)
    for h in range(4):
        pltpu.sync_copy(t_hbm.at[pl.ds(st + h * (_TSH // 4), _TSH // 4)],
                        bounce_v)
        pltpu.sync_copy(bounce_v,
                        tsh.at[pl.ds(st + h * (_TSH // 4), _TSH // 4)])
    plsc.subcore_barrier()

    def chunk(ci, carry):
        base = base0 + ci * _CH
        pltpu.sync_copy(x_hbm.at[pl.ds(base * _H, _CH * _H)], idx_v)
        pltpu.async_copy(tsh.at[idx_v], vals_v, sem).wait()

        def row(r, carry2):
            o = r * _H
            acc = vals_v[pl.ds(o + 184, 16)] * mask_hi
            for k in range(12):
                acc = acc + vals_v[pl.ds(o + k * 16, 16)]
            acc_v[r, :] = acc
            return carry2

        lax.fori_loop(0, _CH, row, 0)
        pltpu.sync_copy(acc_v, p_hbm.at[pl.ds(base, _CH)])
        return carry

    lax.fori_loop(0, _NCH, chunk, 0)


def _final_body(b_ref, p_ref, out_ref):
    s = jnp.sum(p_ref[...], axis=1, keepdims=True)
    z = s * (1.0 / _H) + b_ref[0]
    y = 1.0 / (1.0 + jnp.exp(-z))
    out_ref[...] = jnp.round(y * 10000.0) / 10000.0


def kernel(x, table, W, b):
    t = pl.kernel(
        _t_body,
        out_type=jax.ShapeDtypeStruct((_V,), jnp.float32),
        mesh=plsc.VectorSubcoreMesh(core_axis_name="c", subcore_axis_name="s"),
        compiler_params=pltpu.CompilerParams(needs_layout_passes=False),
        scratch_types=[
            pltpu.VMEM((_D,), jnp.float32),
            pltpu.VMEM((_CHG * 16, _D), jnp.float32),
            pltpu.VMEM((_CHG * 16,), jnp.float32),
            pltpu.SemaphoreType.DMA,
        ],
    )(table, W.reshape(_D))

    part = pl.kernel(
        _gather_body,
        out_type=jax.ShapeDtypeStruct((_B, _D), jnp.float32),
        mesh=plsc.VectorSubcoreMesh(core_axis_name="c", subcore_axis_name="s"),
        compiler_params=pltpu.CompilerParams(needs_layout_passes=False),
        scratch_types=[
            pltpu.VMEM((_CH * _H,), jnp.int32),
            pltpu.VMEM((_CH * _H,), jnp.float32),
            pltpu.VMEM((_CH, _D), jnp.float32),
            pltpu.VMEM((_TSH // 4,), jnp.float32),
            pltpu.VMEM_SHARED((_V,), jnp.float32),
            pltpu.SemaphoreType.DMA,
        ],
    )(x.reshape(_B * _H), t)

    out = pl.pallas_call(
        _final_body,
        grid=(8,),
        in_specs=[
            pl.BlockSpec(memory_space=pltpu.SMEM),
            pl.BlockSpec((_B // 8, _D), lambda i: (i, 0)),
        ],
        out_specs=pl.BlockSpec((_B // 8, 1), lambda i: (i, 0)),
        out_shape=jax.ShapeDtypeStruct((_B, 1), jnp.float32),
    )(b, part)
    return out


# speedup vs baseline: 8.0047x; 1.0517x over previous
import jax
import jax.numpy as jnp
from jax import lax
from jax.experimental import pallas as pl
from jax.experimental.pallas import tpu as pltpu
from jax.experimental.pallas import tpu_sc as plsc

_B = 16384
_H = 200
_D = 16
_V = 1000000
_NC = 2
_NS = 16
_NW = _NC * _NS

_G_TOT = _V // 16
_GPW = 1954
_CHG = 32
_NCHV = 62

_RW = _B // _NW
_CH = 64
_NCH = _RW // _CH


def _t_body(tab_hbm, w_hbm, t_hbm, wv, tab_a, tab_b, skew_v, tv, sem_a, sem_b):
    wid = lax.axis_index("s") * _NC + lax.axis_index("c")
    pltpu.sync_copy(w_hbm, wv)
    wvec = wv[...]
    iota16 = lax.iota(jnp.int32, 16)
    scat = [jj * 16 + ((iota16 + jj) & 15) for jj in range(_D)]
    diag = [iota16 * 16 + ((iota16 + d) & 15) for d in range(_D)]
    nelem = _CHG * 16 * _D

    def flat0(k):
        g0 = jnp.minimum(wid * _GPW + k * _CHG, _G_TOT - _CHG)
        return g0 * 16 * _D

    def process(buf, k):
        r0 = jnp.minimum(wid * _GPW + k * _CHG, _G_TOT - _CHG) * 16

        def group(j, carry2):
            base = j * (16 * _D)
            for jj in range(16):
                m = buf[pl.ds(base + jj * _D, 16)] * wvec
                plsc.store_scatter(skew_v, [scat[jj]], m)
            acc = plsc.load_gather(skew_v, [diag[0]])
            for d in range(1, _D):
                acc = acc + plsc.load_gather(skew_v, [diag[d]])
            tv[pl.ds(j * 16, 16)] = acc
            return carry2

        lax.fori_loop(0, _CHG, group, 0)
        pltpu.sync_copy(tv, t_hbm.at[pl.ds(r0, _CHG * 16)])

    pltpu.async_copy(tab_hbm.at[pl.ds(flat0(0), nelem)], tab_a, sem_a)
    pltpu.async_copy(tab_hbm.at[pl.ds(flat0(1), nelem)], tab_b, sem_b)

    def pair(k2, carry):
        ka = 2 * k2
        pltpu.make_async_copy(tab_hbm.at[pl.ds(0, nelem)], tab_a, sem_a).wait()
        process(tab_a, ka)

        @pl.when(ka + 2 < _NCHV)
        def _():
            pltpu.async_copy(tab_hbm.at[pl.ds(flat0(ka + 2), nelem)],
                             tab_a, sem_a)

        pltpu.make_async_copy(tab_hbm.at[pl.ds(0, nelem)], tab_b, sem_b).wait()
        process(tab_b, ka + 1)

        @pl.when(ka + 3 < _NCHV)
        def _():
            pltpu.async_copy(tab_hbm.at[pl.ds(flat0(ka + 3), nelem)],
                             tab_b, sem_b)

        return carry

    lax.fori_loop(0, _NCHV // 2, pair, 0)


_TSH = 62528


def _gather_body(x_hbm, t_hbm, p_hbm, idx_v, vals_v, acc_v, bounce_v, tsh, sem):
    wid = lax.axis_index("s") * _NC + lax.axis_index("c")
    base0 = wid * _RW
    mask_hi = jnp.where(lax.iota(jnp.int32, 16) >= 8, 1.0, 0.0)

    sid = lax.axis_index("s")
    st = jnp.minimum(sid * _TSH, _V - _TSH)
    for h in range(4):
        pltpu.sync_copy(t_hbm.at[pl.ds(st + h * (_TSH // 4), _TSH // 4)],
                        bounce_v)
        pltpu.sync_copy(bounce_v,
                        tsh.at[pl.ds(st + h * (_TSH // 4), _TSH // 4)])
    plsc.subcore_barrier()

    def chunk(ci, carry):
        base = base0 + ci * _CH
        pltpu.sync_copy(x_hbm.at[pl.ds(base * _H, _CH * _H)], idx_v)
        pltpu.async_copy(tsh.at[idx_v], vals_v, sem).wait()

        def row(r, carry2):
            o = r * _H
            acc = vals_v[pl.ds(o + 184, 16)] * mask_hi
            for k in range(12):
                acc = acc + vals_v[pl.ds(o + k * 16, 16)]
            acc_v[r, :] = acc
            return carry2

        lax.fori_loop(0, _CH, row, 0)
        pltpu.sync_copy(acc_v, p_hbm.at[pl.ds(base, _CH)])
        return carry

    lax.fori_loop(0, _NCH, chunk, 0)


def _final_body(b_ref, p_ref, out_ref):
    s = jnp.sum(p_ref[...], axis=1, keepdims=True)
    z = s * (1.0 / _H) + b_ref[0]
    y = 1.0 / (1.0 + jnp.exp(-z))
    out_ref[...] = jnp.round(y * 10000.0) / 10000.0


def kernel(x, table, W, b):
    t = pl.kernel(
        _t_body,
        out_type=jax.ShapeDtypeStruct((_V,), jnp.float32),
        mesh=plsc.VectorSubcoreMesh(core_axis_name="c", subcore_axis_name="s"),
        compiler_params=pltpu.CompilerParams(needs_layout_passes=False),
        scratch_types=[
            pltpu.VMEM((_D,), jnp.float32),
            pltpu.VMEM((_CHG * 16 * _D,), jnp.float32),
            pltpu.VMEM((_CHG * 16 * _D,), jnp.float32),
            pltpu.VMEM((16 * _D,), jnp.float32),
            pltpu.VMEM((_CHG * 16,), jnp.float32),
            pltpu.SemaphoreType.DMA,
            pltpu.SemaphoreType.DMA,
        ],
    )(table.reshape(_V * _D), W.reshape(_D))

    part = pl.kernel(
        _gather_body,
        out_type=jax.ShapeDtypeStruct((_B, _D), jnp.float32),
        mesh=plsc.VectorSubcoreMesh(core_axis_name="c", subcore_axis_name="s"),
        compiler_params=pltpu.CompilerParams(needs_layout_passes=False),
        scratch_types=[
            pltpu.VMEM((_CH * _H,), jnp.int32),
            pltpu.VMEM((_CH * _H,), jnp.float32),
            pltpu.VMEM((_CH, _D), jnp.float32),
            pltpu.VMEM((_TSH // 4,), jnp.float32),
            pltpu.VMEM_SHARED((_V,), jnp.float32),
            pltpu.SemaphoreType.DMA,
        ],
    )(x.reshape(_B * _H), t)

    out = pl.pallas_call(
        _final_body,
        grid=(8,),
        in_specs=[
            pl.BlockSpec(memory_space=pltpu.SMEM),
            pl.BlockSpec((_B // 8, _D), lambda i: (i, 0)),
        ],
        out_specs=pl.BlockSpec((_B // 8, 1), lambda i: (i, 0)),
        out_shape=jax.ShapeDtypeStruct((_B, 1), jnp.float32),
    )(b, part)
    return out


# speedup vs baseline: 10.0549x; 1.2561x over previous
import jax
import jax.numpy as jnp
from jax import lax
from jax.experimental import pallas as pl
from jax.experimental.pallas import tpu as pltpu
from jax.experimental.pallas import tpu_sc as plsc

_B = 16384
_H = 200
_D = 16
_V = 1000000
_NC = 2
_NS = 16
_NW = _NC * _NS

_G_TOT = _V // 16
_GPW = 1954
_CHG = 16
_NCHV = 124

_RW = _B // _NW
_CH = 64
_NCH = _RW // _CH


def _t_body(tab_hbm, w_hbm, t_hbm, wv, tab_a, tab_b, skew_v, tv, sem_a, sem_b):
    wid = lax.axis_index("s") * _NC + lax.axis_index("c")
    pltpu.sync_copy(w_hbm, wv)
    wvec = wv[...]
    iota16 = lax.iota(jnp.int32, 16)
    scat = [jj * 16 + ((iota16 + jj) & 15) for jj in range(_D)]
    diag = [iota16 * 16 + ((iota16 + d) & 15) for d in range(_D)]
    nrows = _CHG * 16

    def row0(k):
        g0 = jnp.minimum(wid * _GPW + k * _CHG, _G_TOT - _CHG)
        return g0 * 16

    def process(buf, k):
        r0 = row0(k)

        def group(j, carry2):
            base = j * 16
            for jj in range(16):
                m = buf[base + jj, :] * wvec
                plsc.store_scatter(skew_v, [scat[jj]], m)
            acc = plsc.load_gather(skew_v, [diag[0]])
            for d in range(1, _D):
                acc = acc + plsc.load_gather(skew_v, [diag[d]])
            tv[pl.ds(j * 16, 16)] = acc
            return carry2

        lax.fori_loop(0, _CHG, group, 0)
        pltpu.sync_copy(tv, t_hbm.at[pl.ds(r0, _CHG * 16)])

    pltpu.async_copy(tab_hbm.at[pl.ds(row0(0), nrows)], tab_a, sem_a)
    pltpu.async_copy(tab_hbm.at[pl.ds(row0(1), nrows)], tab_b, sem_b)

    def pair(k2, carry):
        ka = 2 * k2
        pltpu.make_async_copy(tab_hbm.at[pl.ds(0, nrows)], tab_a, sem_a).wait()
        process(tab_a, ka)

        @pl.when(ka + 2 < _NCHV)
        def _():
            pltpu.async_copy(tab_hbm.at[pl.ds(row0(ka + 2), nrows)],
                             tab_a, sem_a)

        pltpu.make_async_copy(tab_hbm.at[pl.ds(0, nrows)], tab_b, sem_b).wait()
        process(tab_b, ka + 1)

        @pl.when(ka + 3 < _NCHV)
        def _():
            pltpu.async_copy(tab_hbm.at[pl.ds(row0(ka + 3), nrows)],
                             tab_b, sem_b)

        return carry

    lax.fori_loop(0, _NCHV // 2, pair, 0)


_TSH = 62528


def _gather_body(x_hbm, t_hbm, p_hbm, idx_v, vals_v, acc_v, bounce_v, tsh, sem):
    wid = lax.axis_index("s") * _NC + lax.axis_index("c")
    base0 = wid * _RW
    mask_hi = jnp.where(lax.iota(jnp.int32, 16) >= 8, 1.0, 0.0)

    sid = lax.axis_index("s")
    st = jnp.minimum(sid * _TSH, _V - _TSH)
    for h in range(4):
        pltpu.sync_copy(t_hbm.at[pl.ds(st + h * (_TSH // 4), _TSH // 4)],
                        bounce_v)
        pltpu.sync_copy(bounce_v,
                        tsh.at[pl.ds(st + h * (_TSH // 4), _TSH // 4)])
    plsc.subcore_barrier()

    def chunk(ci, carry):
        base = base0 + ci * _CH
        pltpu.sync_copy(x_hbm.at[pl.ds(base * _H, _CH * _H)], idx_v)
        pltpu.async_copy(tsh.at[idx_v], vals_v, sem).wait()

        def row(r, carry2):
            o = r * _H
            acc = vals_v[pl.ds(o + 184, 16)] * mask_hi
            for k in range(12):
                acc = acc + vals_v[pl.ds(o + k * 16, 16)]
            acc_v[r, :] = acc
            return carry2

        lax.fori_loop(0, _CH, row, 0)
        pltpu.sync_copy(acc_v, p_hbm.at[pl.ds(base, _CH)])
        return carry

    lax.fori_loop(0, _NCH, chunk, 0)


def _final_body(b_ref, p_ref, out_ref):
    s = jnp.sum(p_ref[...], axis=1, keepdims=True)
    z = s * (1.0 / _H) + b_ref[0]
    y = 1.0 / (1.0 + jnp.exp(-z))
    out_ref[...] = jnp.round(y * 10000.0) / 10000.0


def kernel(x, table, W, b):
    t = pl.kernel(
        _t_body,
        out_type=jax.ShapeDtypeStruct((_V,), jnp.float32),
        mesh=plsc.VectorSubcoreMesh(core_axis_name="c", subcore_axis_name="s"),
        compiler_params=pltpu.CompilerParams(needs_layout_passes=False),
        scratch_types=[
            pltpu.VMEM((_D,), jnp.float32),
            pltpu.VMEM((_CHG * 16, _D), jnp.float32),
            pltpu.VMEM((_CHG * 16, _D), jnp.float32),
            pltpu.VMEM((16 * _D,), jnp.float32),
            pltpu.VMEM((_CHG * 16,), jnp.float32),
            pltpu.SemaphoreType.DMA,
            pltpu.SemaphoreType.DMA,
        ],
    )(table, W.reshape(_D))

    part = pl.kernel(
        _gather_body,
        out_type=jax.ShapeDtypeStruct((_B, _D), jnp.float32),
        mesh=plsc.VectorSubcoreMesh(core_axis_name="c", subcore_axis_name="s"),
        compiler_params=pltpu.CompilerParams(needs_layout_passes=False),
        scratch_types=[
            pltpu.VMEM((_CH * _H,), jnp.int32),
            pltpu.VMEM((_CH * _H,), jnp.float32),
            pltpu.VMEM((_CH, _D), jnp.float32),
            pltpu.VMEM((_TSH // 4,), jnp.float32),
            pltpu.VMEM_SHARED((_V,), jnp.float32),
            pltpu.SemaphoreType.DMA,
        ],
    )(x.reshape(_B * _H), t)

    out = pl.pallas_call(
        _final_body,
        grid=(8,),
        in_specs=[
            pl.BlockSpec(memory_space=pltpu.SMEM),
            pl.BlockSpec((_B // 8, _D), lambda i: (i, 0)),
        ],
        out_specs=pl.BlockSpec((_B // 8, 1), lambda i: (i, 0)),
        out_shape=jax.ShapeDtypeStruct((_B, 1), jnp.float32),
    )(b, part)
    return out


# speedup vs baseline: 34.5793x; 3.4390x over previous
import jax
import jax.numpy as jnp
from jax import lax
from jax.experimental import pallas as pl
from jax.experimental.pallas import tpu as pltpu
from jax.experimental.pallas import tpu_sc as plsc

_B = 16384
_H = 200
_D = 16
_V = 1000000
_NC = 2
_NS = 16
_NW = _NC * _NS

_LV = 2048
_CTOT = 487
_NKV = 16
_VTAIL = 999424

_RW = _B // _NW
_CH = 64
_NCH = _RW // _CH


def _t_body(tabT_hbm, tailT_hbm, w_hbm, t_hbm, wv, tab_a, tab_b, tail_v, tv,
            sem_a, sem_b):
    wid = lax.axis_index("s") * _NC + lax.axis_index("c")
    pltpu.sync_copy(w_hbm, wv)
    wvec = wv[...]
    ws = [wvec[d] for d in range(_D)]

    def v0_of(k):
        return jnp.minimum(wid + 32 * k, _CTOT) * _LV

    def start(k, buf, sem):
        pltpu.async_copy(tabT_hbm.at[:, pl.ds(v0_of(k), _LV)], buf, sem)

    def process(buf, k):
        def group(j, carry):
            o = j * 16
            acc = buf[0, pl.ds(o, 16)] * ws[0]
            for d in range(1, _D):
                acc = acc + buf[d, pl.ds(o, 16)] * ws[d]
            tv[pl.ds(o, 16)] = acc
            return carry

        lax.fori_loop(0, _LV // 16, group, 0)
        pltpu.sync_copy(tv, t_hbm.at[pl.ds(v0_of(k), _LV)])

    start(0, tab_a, sem_a)
    start(1, tab_b, sem_b)

    def pair(k2, carry):
        ka = 2 * k2
        pltpu.make_async_copy(tabT_hbm.at[:, pl.ds(0, _LV)], tab_a,
                              sem_a).wait()
        process(tab_a, ka)

        @pl.when(ka + 2 < _NKV)
        def _():
            start(ka + 2, tab_a, sem_a)

        pltpu.make_async_copy(tabT_hbm.at[:, pl.ds(0, _LV)], tab_b,
                              sem_b).wait()
        process(tab_b, ka + 1)

        @pl.when(ka + 3 < _NKV)
        def _():
            start(ka + 3, tab_b, sem_b)

        return carry

    lax.fori_loop(0, _NKV // 2, pair, 0)

    @pl.when(wid == 0)
    def _():
        pltpu.sync_copy(tailT_hbm, tail_v)

        def tgroup(j, carry):
            o = j * 16
            acc = tail_v[0, pl.ds(o, 16)] * ws[0]
            for d in range(1, _D):
                acc = acc + tail_v[d, pl.ds(o, 16)] * ws[d]
            tv[pl.ds(o, 16)] = acc
            return carry

        lax.fori_loop(0, 576 // 16, tgroup, 0)
        pltpu.sync_copy(tv.at[pl.ds(0, 576)], t_hbm.at[pl.ds(_VTAIL, 576)])


_TSH = 62528


def _gather_body(x_hbm, t_hbm, p_hbm, idx_v, vals_v, acc_v, bounce_v, tsh, sem):
    wid = lax.axis_index("s") * _NC + lax.axis_index("c")
    base0 = wid * _RW
    mask_hi = jnp.where(lax.iota(jnp.int32, 16) >= 8, 1.0, 0.0)

    sid = lax.axis_index("s")
    st = jnp.minimum(sid * _TSH, _V - _TSH)
    for h in range(4):
        pltpu.sync_copy(t_hbm.at[pl.ds(st + h * (_TSH // 4), _TSH // 4)],
                        bounce_v)
        pltpu.sync_copy(bounce_v,
                        tsh.at[pl.ds(st + h * (_TSH // 4), _TSH // 4)])
    plsc.subcore_barrier()

    def chunk(ci, carry):
        base = base0 + ci * _CH
        pltpu.sync_copy(x_hbm.at[pl.ds(base * _H, _CH * _H)], idx_v)
        pltpu.async_copy(tsh.at[idx_v], vals_v, sem).wait()

        def row(r, carry2):
            o = r * _H
            acc = vals_v[pl.ds(o + 184, 16)] * mask_hi
            for k in range(12):
                acc = acc + vals_v[pl.ds(o + k * 16, 16)]
            acc_v[r, :] = acc
            return carry2

        lax.fori_loop(0, _CH, row, 0)
        pltpu.sync_copy(acc_v, p_hbm.at[pl.ds(base, _CH)])
        return carry

    lax.fori_loop(0, _NCH, chunk, 0)


def _final_body(b_ref, p_ref, out_ref):
    s = jnp.sum(p_ref[...], axis=1, keepdims=True)
    z = s * (1.0 / _H) + b_ref[0]
    y = 1.0 / (1.0 + jnp.exp(-z))
    out_ref[...] = jnp.round(y * 10000.0) / 10000.0


def kernel(x, table, W, b):
    t = pl.kernel(
        _t_body,
        out_type=jax.ShapeDtypeStruct((_V,), jnp.float32),
        mesh=plsc.VectorSubcoreMesh(core_axis_name="c", subcore_axis_name="s"),
        compiler_params=pltpu.CompilerParams(
            needs_layout_passes=False, use_tc_tiling_on_sc=True),
        scratch_types=[
            pltpu.VMEM((_D,), jnp.float32),
            pltpu.VMEM((_D, _LV), jnp.float32),
            pltpu.VMEM((_D, _LV), jnp.float32),
            pltpu.VMEM((_D, _V - _VTAIL), jnp.float32),
            pltpu.VMEM((_LV,), jnp.float32),
            pltpu.SemaphoreType.DMA,
            pltpu.SemaphoreType.DMA,
        ],
    )(table.T, table[_VTAIL:, :].T, W.reshape(_D))

    part = pl.kernel(
        _gather_body,
        out_type=jax.ShapeDtypeStruct((_B, _D), jnp.float32),
        mesh=plsc.VectorSubcoreMesh(core_axis_name="c", subcore_axis_name="s"),
        compiler_params=pltpu.CompilerParams(needs_layout_passes=False),
        scratch_types=[
            pltpu.VMEM((_CH * _H,), jnp.int32),
            pltpu.VMEM((_CH * _H,), jnp.float32),
            pltpu.VMEM((_CH, _D), jnp.float32),
            pltpu.VMEM((_TSH // 4,), jnp.float32),
            pltpu.VMEM_SHARED((_V,), jnp.float32),
            pltpu.SemaphoreType.DMA,
        ],
    )(x.reshape(_B * _H), t)

    out = pl.pallas_call(
        _final_body,
        grid=(8,),
        in_specs=[
            pl.BlockSpec(memory_space=pltpu.SMEM),
            pl.BlockSpec((_B // 8, _D), lambda i: (i, 0)),
        ],
        out_specs=pl.BlockSpec((_B // 8, 1), lambda i: (i, 0)),
        out_shape=jax.ShapeDtypeStruct((_B, 1), jnp.float32),
    )(b, part)
    return out
